# baseline (speedup 1.0000x reference)
import jax
import jax.numpy as jnp
from jax.experimental import pallas as pl
from jax.experimental.pallas import tpu as pltpu

_TN = 1024


def _matmul_kernel(x_ref, mem_ref, out_ref):
    xb = x_ref[...].astype(jnp.bfloat16)
    mb = mem_ref[...].astype(jnp.bfloat16)
    out_ref[...] = jax.lax.dot_general(
        xb, mb, (((1,), (1,)), ((), ())),
        preferred_element_type=jnp.float32)


def kernel(x, targets, memory):
    del targets
    b, d = x.shape
    n = memory.shape[0]
    return pl.pallas_call(
        _matmul_kernel,
        grid=(n // _TN,),
        in_specs=[
            pl.BlockSpec((b, d), lambda j: (0, 0)),
            pl.BlockSpec((_TN, d), lambda j: (j, 0)),
        ],
        out_specs=pl.BlockSpec((b, _TN), lambda j: (0, j)),
        out_shape=jax.ShapeDtypeStruct((b, n), jnp.float32),
        compiler_params=pltpu.CompilerParams(
            dimension_semantics=("parallel",)),
    )(x, memory)


# speedup vs baseline: 1.0926x; 1.0926x over previous
import jax
import jax.numpy as jnp
from jax.experimental import pallas as pl
from jax.experimental.pallas import tpu as pltpu

_TN = 2048


def _matmul_kernel(x_ref, mem_ref, out_ref):
    xb = x_ref[...].astype(jnp.bfloat16)
    mb = mem_ref[...].astype(jnp.bfloat16)
    out_ref[...] = jax.lax.dot_general(
        xb, mb, (((1,), (1,)), ((), ())),
        preferred_element_type=jnp.float32)


def kernel(x, targets, memory):
    del targets
    b, d = x.shape
    n = memory.shape[0]
    return pl.pallas_call(
        _matmul_kernel,
        grid=(n // _TN,),
        in_specs=[
            pl.BlockSpec((b, d), lambda j: (0, 0)),
            pl.BlockSpec((_TN, d), lambda j: (j, 0)),
        ],
        out_specs=pl.BlockSpec((b, _TN), lambda j: (0, j)),
        out_shape=jax.ShapeDtypeStruct((b, n), jnp.float32),
        compiler_params=pltpu.CompilerParams(
            dimension_semantics=("parallel",)),
    )(x, memory)


# speedup vs baseline: 1.1240x; 1.0287x over previous
import jax
import jax.numpy as jnp
from jax.experimental import pallas as pl
from jax.experimental.pallas import tpu as pltpu

_TN = 4096


def _matmul_kernel(x_ref, mem_ref, out_ref):
    xb = x_ref[...].astype(jnp.bfloat16)
    mb = mem_ref[...].astype(jnp.bfloat16)
    out_ref[...] = jax.lax.dot_general(
        xb, mb, (((1,), (1,)), ((), ())),
        preferred_element_type=jnp.float32)


def kernel(x, targets, memory):
    del targets
    b, d = x.shape
    n = memory.shape[0]
    return pl.pallas_call(
        _matmul_kernel,
        grid=(n // _TN,),
        in_specs=[
            pl.BlockSpec((b, d), lambda j: (0, 0)),
            pl.BlockSpec((_TN, d), lambda j: (j, 0)),
        ],
        out_specs=pl.BlockSpec((b, _TN), lambda j: (0, j)),
        out_shape=jax.ShapeDtypeStruct((b, n), jnp.float32),
        compiler_params=pltpu.CompilerParams(
            dimension_semantics=("parallel",)),
    )(x, memory)
